# baseline (speedup 1.0000x reference)
import jax
import jax.numpy as jnp
from jax.experimental import pallas as pl
from jax.experimental.pallas import tpu as pltpu

_BM = 400


def _proj_kernel(x_ref, w_ref, out_ref):
    out_ref[...] = jnp.dot(x_ref[...], w_ref[...],
                           preferred_element_type=jnp.float32)


def _layer1_kernel(adj_ref, s_ref, b1_ref, w2_ref, out_ref):
    h = jnp.dot(adj_ref[...], s_ref[...], preferred_element_type=jnp.float32)
    h = jnp.maximum(h + b1_ref[...], 0.0)
    out_ref[...] = jnp.dot(h, w2_ref[...], preferred_element_type=jnp.float32)


def _layer2_kernel(adj_ref, s_ref, b2_ref, out_ref):
    acc = jnp.dot(adj_ref[...], s_ref[...], preferred_element_type=jnp.float32)
    out_ref[...] = acc + b2_ref[...]


def kernel(x, adj, W1, b1, W2, b2):
    N, nfeat = x.shape
    nhid = W1.shape[1]
    nout = W2.shape[1]

    s1 = pl.pallas_call(
        _proj_kernel,
        out_shape=jax.ShapeDtypeStruct((N, nhid), jnp.float32),
    )(x, W1)

    grid = (N // _BM,)
    params = pltpu.CompilerParams(dimension_semantics=("arbitrary",))

    s2 = pl.pallas_call(
        _layer1_kernel,
        grid=grid,
        in_specs=[
            pl.BlockSpec((_BM, N), lambda i: (i, 0)),
            pl.BlockSpec((N, nhid), lambda i: (0, 0)),
            pl.BlockSpec((1, nhid), lambda i: (0, 0)),
            pl.BlockSpec((nhid, nout), lambda i: (0, 0)),
        ],
        out_specs=pl.BlockSpec((_BM, nout), lambda i: (i, 0)),
        out_shape=jax.ShapeDtypeStruct((N, nout), jnp.float32),
        compiler_params=params,
    )(adj, s1, b1.reshape(1, nhid), W2)

    out = pl.pallas_call(
        _layer2_kernel,
        grid=grid,
        in_specs=[
            pl.BlockSpec((_BM, N), lambda i: (i, 0)),
            pl.BlockSpec((N, nout), lambda i: (0, 0)),
            pl.BlockSpec((1, nout), lambda i: (0, 0)),
        ],
        out_specs=pl.BlockSpec((_BM, nout), lambda i: (i, 0)),
        out_shape=jax.ShapeDtypeStruct((N, nout), jnp.float32),
        compiler_params=params,
    )(adj, s2, b2.reshape(1, nout))

    return out
